# baseline (speedup 1.0000x reference)
import functools

import jax
import jax.numpy as jnp
from jax import lax
from jax.experimental import pallas as pl
from jax.experimental.pallas import tpu as pltpu
from jax.experimental.pallas import tpu_sc as plsc

_LANES = 16
_RING = 16
_BLK = 128


def _make_colgather(V, D, B):
  info = plsc.get_sparse_core_info()
  NC, NS = info.num_cores, info.num_subcores
  NW = NC * NS
  assert B % (_BLK * NW) == 0 and D == 32
  b_per_w = B // NW
  n_blocks = b_per_w // _BLK
  mesh = plsc.VectorSubcoreMesh(core_axis_name="c", subcore_axis_name="s")

  @functools.partial(
      pl.kernel,
      mesh=mesh,
      compiler_params=pltpu.CompilerParams(needs_layout_passes=False),
      out_type=jax.ShapeDtypeStruct((D, B), jnp.float32),
      scratch_types=[
          pltpu.VMEM((b_per_w,), jnp.int32),
          [pltpu.VMEM((D, 128), jnp.float32) for _ in range(_RING)],
          [pltpu.VMEM((D, _BLK), jnp.float32) for _ in range(2)],
          [pltpu.SemaphoreType.DMA for _ in range(_RING)],
          [pltpu.SemaphoreType.DMA for _ in range(2)],
      ],
  )
  def gather_kernel(w_hbm, idx_hbm, out_hbm, idx_v, wins, slabs, sems, osems):
    wid = lax.axis_index("s") * NC + lax.axis_index("c")
    base = wid * b_per_w
    pltpu.sync_copy(idx_hbm.at[pl.ds(base, b_per_w)], idx_v)
    lanes = lax.iota(jnp.int32, _LANES)

    def load_group(g):
      pos = jnp.minimum(g * _RING + lanes, b_per_w - 1)
      return plsc.load_gather(idx_v, [pos])

    def fire(s, c):
      c0 = pl.multiple_of((c // 128) * 128, 128)
      pltpu.async_copy(w_hbm.at[:, pl.ds(c0, 128)], wins[s], sems[s])

    def drain(s, c, bcol, slab):
      pltpu.make_async_copy(
          w_hbm.at[:, pl.ds(0, 128)], wins[s], sems[s]
      ).wait()
      off = jnp.full((_LANES,), c % 128, jnp.int32)
      col = jnp.full((_LANES,), bcol, jnp.int32)
      lo = plsc.load_gather(wins[s], [lanes, off])
      hi = plsc.load_gather(wins[s], [_LANES + lanes, off])
      plsc.store_scatter(slab, [lanes, col], lo)
      plsc.store_scatter(slab, [_LANES + lanes, col], hi)

    groups_per_blk = _BLK // _RING
    cvec0 = load_group(0)
    for s in range(_RING):
      fire(s, cvec0[s])

    for k in range(n_blocks):
      slab = slabs[k % 2]
      if k >= 2:
        pltpu.make_async_copy(
            slab, out_hbm.at[:, pl.ds(base, _BLK)], osems[k % 2]
        ).wait()

      def body(i, cvec):
        g = k * groups_per_blk + i
        nxt = load_group(g + 1)
        for s in range(_RING):
          drain(s, cvec[s], (i * _RING + s) % _BLK, slab)

          @pl.when(g + 1 < n_blocks * groups_per_blk)
          def _():
            fire(s, nxt[s])

        return nxt

      cvec0 = lax.fori_loop(0, groups_per_blk, body, cvec0)
      pltpu.async_copy(
          slab, out_hbm.at[:, pl.ds(base + k * _BLK, _BLK)], osems[k % 2]
      )

    for k in (n_blocks - 2, n_blocks - 1):
      pltpu.make_async_copy(
          slabs[k % 2], out_hbm.at[:, pl.ds(base, _BLK)], osems[k % 2]
      ).wait()

  return gather_kernel


def kernel(x, weight):
  V, D = weight.shape
  B = x.shape[0]
  out_t = _make_colgather(V, D, B)(weight.T, x.astype(jnp.int32))
  return out_t.T
